# baseline (speedup 1.0000x reference)
import functools

import jax
import jax.numpy as jnp
from jax import lax
from jax.experimental import pallas as pl
from jax.experimental.pallas import tpu as pltpu
from jax.experimental.pallas import tpu_sc as plsc

B = 16384
V = 100000
D_ID = 64
D_F = 8
NF = 12
NCOL = 1 + NF

NC = 2
NS = 16
NW = NC * NS
L = 16

CPW = B // NW
NB = 32
CB = CPW // NB
FPB = CB * NF
ICH = 96

FROWS = NF * V // 16


def _nes_body(uf_hbm, if_hbm, uidb_tab, ufeat_tab, iidb_tab, ifeat_tab,
              out_hbm, uf_v, if_v,
              uid_idx0, iid_idx0, ufidx0, ifidx0, uoff0, ioff0,
              uid_rows0, iid_rows0, uf_rows0, if_rows0,
              uid_idx1, iid_idx1, ufidx1, ifidx1, uoff1, ioff1,
              uid_rows1, iid_rows1, uf_rows1, if_rows1,
              out_v, sem0, sem1):
    wid = lax.axis_index("s") * NC + lax.axis_index("c")
    base = wid * CPW

    iota = lax.iota(jnp.int32, L)
    zeros = iota * 0
    i_d8 = iota // 8
    i_m8 = iota % 8
    m15 = iota == 15

    bufs = [
        (uid_idx0, iid_idx0, ufidx0, ifidx0, uoff0, ioff0,
         uid_rows0, iid_rows0, uf_rows0, if_rows0, sem0),
        (uid_idx1, iid_idx1, ufidx1, ifidx1, uoff1, ioff1,
         uid_rows1, iid_rows1, uf_rows1, if_rows1, sem1),
    ]

    pltpu.sync_copy(uf_hbm.at[pl.ds(base, CPW)], uf_v)
    pltpu.sync_copy(if_hbm.at[pl.ds(base, CPW)], if_v)

    def build_and_fire(g):
        (uid_idx, iid_idx, ufidx, ifidx, uoff, ioff,
         uid_rows, iid_rows, uf_rows, if_rows, sem) = bufs[g % 2]
        e0 = g * CB

        r = e0 + iota
        uid_idx[pl.ds(0, L)] = plsc.load_gather(uf_v, [r, zeros])
        iid_idx[pl.ds(0, L)] = plsc.load_gather(if_v, [r, zeros])

        def f_idx_body(t, _):
            k = t * L + iota
            e = k // NF
            i = k - e * NF
            fr = plsc.load_gather(uf_v, [e0 + e, 1 + i]) + i * V
            fi = plsc.load_gather(if_v, [e0 + e, 1 + i]) + i * V
            sl = pl.ds(t * L, L)
            ufidx[sl] = fr >> 4
            uoff[sl] = (fr & 15) * 8
            ifidx[sl] = fi >> 4
            ioff[sl] = (fi & 15) * 8
            return 0

        lax.fori_loop(0, FPB // L, f_idx_body, 0, unroll=2)

        copies = [
            pltpu.async_copy(uidb_tab.at[uid_idx], uid_rows, sem),
            pltpu.async_copy(iidb_tab.at[iid_idx], iid_rows, sem),
        ]
        for j in range(FPB // ICH):
            sl = pl.ds(j * ICH, ICH)
            copies.append(
                pltpu.async_copy(ufeat_tab.at[ufidx.at[sl]], uf_rows.at[sl], sem))
            copies.append(
                pltpu.async_copy(ifeat_tab.at[ifidx.at[sl]], if_rows.at[sl], sem))
        return copies

    def compute(g):
        (uid_idx, iid_idx, ufidx, ifidx, uoff, ioff,
         uid_rows, iid_rows, uf_rows, if_rows, sem) = bufs[g % 2]
        e0 = g * CB

        def dot_body(e, _):
            acc = uid_rows[e, pl.ds(0, L)] * iid_rows[e, pl.ds(0, L)]
            for s in range(1, D_ID // L):
                acc += uid_rows[e, pl.ds(s * L, L)] * iid_rows[e, pl.ds(s * L, L)]
            acc += uid_rows[e, pl.ds(D_ID, L)] + iid_rows[e, pl.ds(D_ID, L)]
            rb = e * NF
            for s in range(NF * D_F // L):
                row = rb + 2 * s + i_d8
                cu = plsc.load_gather(uoff, [row]) + i_m8
                ci = plsc.load_gather(ioff, [row]) + i_m8
                pu = plsc.load_gather(uf_rows, [row, cu])
                pv = plsc.load_gather(if_rows, [row, ci])
                acc += pu * pv
            tot = plsc.cumsum(acc)
            plsc.store_scatter(out_v, [zeros + (e0 + e)], tot, mask=m15)
            return 0

        lax.fori_loop(0, CB, dot_body, 0)

    copies_prev = build_and_fire(0)
    for g in range(NB):
        copies_next = build_and_fire(g + 1) if g + 1 < NB else None
        for c in copies_prev:
            c.wait()
        compute(g)
        copies_prev = copies_next

    pltpu.sync_copy(out_v, out_hbm.at[pl.ds(base, CPW)])


@functools.partial(jax.jit, static_argnames=())
def kernel(users_features, items_features, user_id_table, user_feat_tables,
           user_bias, item_id_table, item_feat_tables, item_bias):
    ufeat128 = user_feat_tables.reshape(FROWS, 128)
    ifeat128 = item_feat_tables.reshape(FROWS, 128)
    zpad = jnp.zeros((user_id_table.shape[0], 128 - D_ID - 1), jnp.float32)
    uidb = jnp.concatenate([user_id_table, user_bias, zpad], axis=1)
    iidb = jnp.concatenate([item_id_table, item_bias, zpad], axis=1)
    mesh = plsc.VectorSubcoreMesh(core_axis_name="c", subcore_axis_name="s")
    per_parity = [
        pltpu.VMEM((CB,), jnp.int32),
        pltpu.VMEM((CB,), jnp.int32),
        pltpu.VMEM((FPB,), jnp.int32),
        pltpu.VMEM((FPB,), jnp.int32),
        pltpu.VMEM((FPB,), jnp.int32),
        pltpu.VMEM((FPB,), jnp.int32),
        pltpu.VMEM((CB, 128), jnp.float32),
        pltpu.VMEM((CB, 128), jnp.float32),
        pltpu.VMEM((FPB, 128), jnp.float32),
        pltpu.VMEM((FPB, 128), jnp.float32),
    ]
    f = pl.kernel(
        _nes_body,
        out_type=jax.ShapeDtypeStruct((B,), jnp.float32),
        mesh=mesh,
        scratch_types=(
            [pltpu.VMEM((CPW, NCOL), jnp.int32),
             pltpu.VMEM((CPW, NCOL), jnp.int32)]
            + per_parity + per_parity
            + [pltpu.VMEM((CPW,), jnp.float32),
               pltpu.SemaphoreType.DMA,
               pltpu.SemaphoreType.DMA]
        ),
        compiler_params=pltpu.CompilerParams(
            needs_layout_passes=False, use_tc_tiling_on_sc=False),
    )
    return f(users_features, items_features, uidb, ufeat128, iidb, ifeat128)


# speedup vs baseline: 1.0535x; 1.0535x over previous
import functools

import jax
import jax.numpy as jnp
from jax import lax
from jax.experimental import pallas as pl
from jax.experimental.pallas import tpu as pltpu
from jax.experimental.pallas import tpu_sc as plsc

B = 16384
V = 100000
D_ID = 64
D_F = 8
NF = 12
NCOL = 1 + NF

NC = 2
NS = 16
NW = NC * NS
L = 16

CPW = B // NW
NB = 32
CB = CPW // NB
FPB = CB * NF
ICH = 96

FROWS = NF * V // 16


def _nes_body(uf_hbm, if_hbm, uid_tab, ub_tab, ufeat_tab,
              iid_tab, ib_tab, ifeat_tab,
              out_hbm, uf_v, if_v,
              uid_idx0, iid_idx0, ubidx0, ibidx0, uboff0, iboff0,
              ufidx0, ifidx0, uoff0, ioff0,
              uid_rows0, iid_rows0, ub_rows0, ib_rows0, uf_rows0, if_rows0,
              uid_idx1, iid_idx1, ubidx1, ibidx1, uboff1, iboff1,
              ufidx1, ifidx1, uoff1, ioff1,
              uid_rows1, iid_rows1, ub_rows1, ib_rows1, uf_rows1, if_rows1,
              bsum, out_v, sem0, sem1):
    wid = lax.axis_index("s") * NC + lax.axis_index("c")
    base = wid * CPW

    iota = lax.iota(jnp.int32, L)
    zeros = iota * 0
    i_d8 = iota // 8
    i_m8 = iota % 8
    m15 = iota == 15

    bufs = [
        (uid_idx0, iid_idx0, ubidx0, ibidx0, uboff0, iboff0,
         ufidx0, ifidx0, uoff0, ioff0,
         uid_rows0, iid_rows0, ub_rows0, ib_rows0, uf_rows0, if_rows0, sem0),
        (uid_idx1, iid_idx1, ubidx1, ibidx1, uboff1, iboff1,
         ufidx1, ifidx1, uoff1, ioff1,
         uid_rows1, iid_rows1, ub_rows1, ib_rows1, uf_rows1, if_rows1, sem1),
    ]

    pltpu.sync_copy(uf_hbm.at[pl.ds(base, CPW)], uf_v)
    pltpu.sync_copy(if_hbm.at[pl.ds(base, CPW)], if_v)

    def build_and_fire(g):
        (uid_idx, iid_idx, ubidx, ibidx, uboff, iboff,
         ufidx, ifidx, uoff, ioff,
         uid_rows, iid_rows, ub_rows, ib_rows, uf_rows, if_rows, sem) = bufs[g % 2]
        e0 = g * CB

        r = e0 + iota
        u = plsc.load_gather(uf_v, [r, zeros])
        v = plsc.load_gather(if_v, [r, zeros])
        uid_idx[pl.ds(0, L)] = u
        iid_idx[pl.ds(0, L)] = v
        ubidx[pl.ds(0, L)] = u >> 7
        uboff[pl.ds(0, L)] = u & 127
        ibidx[pl.ds(0, L)] = v >> 7
        iboff[pl.ds(0, L)] = v & 127

        def f_idx_body(t, _):
            k = t * L + iota
            e = k // NF
            i = k - e * NF
            fr = plsc.load_gather(uf_v, [e0 + e, 1 + i]) + i * V
            fi = plsc.load_gather(if_v, [e0 + e, 1 + i]) + i * V
            sl = pl.ds(t * L, L)
            ufidx[sl] = fr >> 4
            uoff[sl] = (fr & 15) * 8
            ifidx[sl] = fi >> 4
            ioff[sl] = (fi & 15) * 8
            return 0

        lax.fori_loop(0, FPB // L, f_idx_body, 0, unroll=2)

        copies = [
            pltpu.async_copy(uid_tab.at[uid_idx], uid_rows, sem),
            pltpu.async_copy(iid_tab.at[iid_idx], iid_rows, sem),
            pltpu.async_copy(ub_tab.at[ubidx], ub_rows, sem),
            pltpu.async_copy(ib_tab.at[ibidx], ib_rows, sem),
        ]
        for j in range(FPB // ICH):
            sl = pl.ds(j * ICH, ICH)
            copies.append(
                pltpu.async_copy(ufeat_tab.at[ufidx.at[sl]], uf_rows.at[sl], sem))
            copies.append(
                pltpu.async_copy(ifeat_tab.at[ifidx.at[sl]], if_rows.at[sl], sem))
        return copies

    def compute(g):
        (uid_idx, iid_idx, ubidx, ibidx, uboff, iboff,
         ufidx, ifidx, uoff, ioff,
         uid_rows, iid_rows, ub_rows, ib_rows, uf_rows, if_rows, sem) = bufs[g % 2]
        e0 = g * CB

        bsum[pl.ds(0, CB)] = (
            plsc.load_gather(ub_rows, [iota, uboff[pl.ds(0, L)]])
            + plsc.load_gather(ib_rows, [iota, iboff[pl.ds(0, L)]]))

        def dot_body(e, _):
            acc = uid_rows[e, pl.ds(0, L)] * iid_rows[e, pl.ds(0, L)]
            for s in range(1, D_ID // L):
                acc += uid_rows[e, pl.ds(s * L, L)] * iid_rows[e, pl.ds(s * L, L)]
            rb = e * NF
            for s in range(NF * D_F // L):
                row = rb + 2 * s + i_d8
                cu = plsc.load_gather(uoff, [row]) + i_m8
                ci = plsc.load_gather(ioff, [row]) + i_m8
                pu = plsc.load_gather(uf_rows, [row, cu])
                pv = plsc.load_gather(if_rows, [row, ci])
                acc += pu * pv
            tot = plsc.cumsum(acc) + plsc.load_gather(bsum, [zeros + e])
            plsc.store_scatter(out_v, [zeros + (e0 + e)], tot, mask=m15)
            return 0

        lax.fori_loop(0, CB, dot_body, 0)

    copies_prev = build_and_fire(0)
    for g in range(NB):
        copies_next = build_and_fire(g + 1) if g + 1 < NB else None
        for c in copies_prev:
            c.wait()
        compute(g)
        copies_prev = copies_next

    pltpu.sync_copy(out_v, out_hbm.at[pl.ds(base, CPW)])


@functools.partial(jax.jit, static_argnames=())
def kernel(users_features, items_features, user_id_table, user_feat_tables,
           user_bias, item_id_table, item_feat_tables, item_bias):
    ufeat128 = user_feat_tables.reshape(FROWS, 128)
    ifeat128 = item_feat_tables.reshape(FROWS, 128)
    brows = (V + 127) // 128
    pad = brows * 128 - user_bias.shape[0]
    ub128 = jnp.pad(user_bias.reshape(-1), (0, pad)).reshape(brows, 128)
    ib128 = jnp.pad(item_bias.reshape(-1), (0, pad)).reshape(brows, 128)
    mesh = plsc.VectorSubcoreMesh(core_axis_name="c", subcore_axis_name="s")
    per_parity = [
        pltpu.VMEM((CB,), jnp.int32),
        pltpu.VMEM((CB,), jnp.int32),
        pltpu.VMEM((CB,), jnp.int32),
        pltpu.VMEM((CB,), jnp.int32),
        pltpu.VMEM((CB,), jnp.int32),
        pltpu.VMEM((CB,), jnp.int32),
        pltpu.VMEM((FPB,), jnp.int32),
        pltpu.VMEM((FPB,), jnp.int32),
        pltpu.VMEM((FPB,), jnp.int32),
        pltpu.VMEM((FPB,), jnp.int32),
        pltpu.VMEM((CB, D_ID), jnp.float32),
        pltpu.VMEM((CB, D_ID), jnp.float32),
        pltpu.VMEM((CB, 128), jnp.float32),
        pltpu.VMEM((CB, 128), jnp.float32),
        pltpu.VMEM((FPB, 128), jnp.float32),
        pltpu.VMEM((FPB, 128), jnp.float32),
    ]
    f = pl.kernel(
        _nes_body,
        out_type=jax.ShapeDtypeStruct((B,), jnp.float32),
        mesh=mesh,
        scratch_types=(
            [pltpu.VMEM((CPW, NCOL), jnp.int32),
             pltpu.VMEM((CPW, NCOL), jnp.int32)]
            + per_parity + per_parity
            + [pltpu.VMEM((CB,), jnp.float32),
               pltpu.VMEM((CPW,), jnp.float32),
               pltpu.SemaphoreType.DMA,
               pltpu.SemaphoreType.DMA]
        ),
        compiler_params=pltpu.CompilerParams(
            needs_layout_passes=False, use_tc_tiling_on_sc=False),
    )
    return f(users_features, items_features, user_id_table, ub128,
             ufeat128, item_id_table, ib128, ifeat128)


# speedup vs baseline: 1.1185x; 1.0617x over previous
import functools

import jax
import jax.numpy as jnp
from jax import lax
from jax.experimental import pallas as pl
from jax.experimental.pallas import tpu as pltpu
from jax.experimental.pallas import tpu_sc as plsc

B = 16384
V = 100000
D_ID = 64
D_F = 8
NF = 12
NCOL = 1 + NF

NC = 2
NS = 16
NW = NC * NS
L = 16

CPW = B // NW
NB = 32
CB = CPW // NB
FPB = CB * NF
ICH = 96

FROWS = NF * V // 16


def _nes_body(uf_hbm, if_hbm, uid_tab, ub_tab, ufeat_tab,
              iid_tab, ib_tab, ifeat_tab,
              out_hbm, uf_v, if_v,
              uid_idx0, iid_idx0, ubidx0, ibidx0, uboff0, iboff0,
              ufidx0, ifidx0, uoff0, ioff0,
              uid_rows0, iid_rows0, ub_rows0, ib_rows0, uf_rows0, if_rows0,
              uid_idx1, iid_idx1, ubidx1, ibidx1, uboff1, iboff1,
              ufidx1, ifidx1, uoff1, ioff1,
              uid_rows1, iid_rows1, ub_rows1, ib_rows1, uf_rows1, if_rows1,
              bsum, out_v, sem0, sem1):
    wid = lax.axis_index("s") * NC + lax.axis_index("c")
    base = wid * CPW

    iota = lax.iota(jnp.int32, L)
    zeros = iota * 0
    i_d8 = iota // 8
    i_m8 = iota % 8
    m15 = iota == 15

    bufs = [
        (uid_idx0, iid_idx0, ubidx0, ibidx0, uboff0, iboff0,
         ufidx0, ifidx0, uoff0, ioff0,
         uid_rows0, iid_rows0, ub_rows0, ib_rows0, uf_rows0, if_rows0, sem0),
        (uid_idx1, iid_idx1, ubidx1, ibidx1, uboff1, iboff1,
         ufidx1, ifidx1, uoff1, ioff1,
         uid_rows1, iid_rows1, ub_rows1, ib_rows1, uf_rows1, if_rows1, sem1),
    ]

    pltpu.sync_copy(uf_hbm.at[pl.ds(base, CPW)], uf_v)
    pltpu.sync_copy(if_hbm.at[pl.ds(base, CPW)], if_v)

    def build_and_fire(g):
        (uid_idx, iid_idx, ubidx, ibidx, uboff, iboff,
         ufidx, ifidx, uoff, ioff,
         uid_rows, iid_rows, ub_rows, ib_rows, uf_rows, if_rows, sem) = bufs[g % 2]
        e0 = g * CB

        r = e0 + iota
        u = plsc.load_gather(uf_v, [r, zeros])
        v = plsc.load_gather(if_v, [r, zeros])
        uid_idx[pl.ds(0, L)] = u
        iid_idx[pl.ds(0, L)] = v
        ubidx[pl.ds(0, L)] = u >> 7
        uboff[pl.ds(0, L)] = u & 127
        ibidx[pl.ds(0, L)] = v >> 7
        iboff[pl.ds(0, L)] = v & 127

        def f_idx_body(t, _):
            k = t * L + iota
            e = k // NF
            i = k - e * NF
            fr = plsc.load_gather(uf_v, [e0 + e, 1 + i]) + i * V
            fi = plsc.load_gather(if_v, [e0 + e, 1 + i]) + i * V
            sl = pl.ds(t * L, L)
            ufidx[sl] = fr
            ifidx[sl] = fi
            return 0

        lax.fori_loop(0, FPB // L, f_idx_body, 0, unroll=2)

        copies = [
            pltpu.async_copy(uid_tab.at[uid_idx], uid_rows, sem),
            pltpu.async_copy(iid_tab.at[iid_idx], iid_rows, sem),
            pltpu.async_copy(ub_tab.at[ubidx], ub_rows, sem),
            pltpu.async_copy(ib_tab.at[ibidx], ib_rows, sem),
        ]
        for j in range(FPB // ICH):
            sl = pl.ds(j * ICH, ICH)
            copies.append(
                pltpu.async_copy(ufeat_tab.at[ufidx.at[sl]], uf_rows.at[sl], sem))
            copies.append(
                pltpu.async_copy(ifeat_tab.at[ifidx.at[sl]], if_rows.at[sl], sem))
        return copies

    def compute(g):
        (uid_idx, iid_idx, ubidx, ibidx, uboff, iboff,
         ufidx, ifidx, uoff, ioff,
         uid_rows, iid_rows, ub_rows, ib_rows, uf_rows, if_rows, sem) = bufs[g % 2]
        e0 = g * CB

        bsum[pl.ds(0, CB)] = (
            plsc.load_gather(ub_rows, [iota, uboff[pl.ds(0, L)]])
            + plsc.load_gather(ib_rows, [iota, iboff[pl.ds(0, L)]]))

        def dot_body(e, _):
            acc = uid_rows[e, pl.ds(0, L)] * iid_rows[e, pl.ds(0, L)]
            for s in range(1, D_ID // L):
                acc += uid_rows[e, pl.ds(s * L, L)] * iid_rows[e, pl.ds(s * L, L)]
            rb = e * NF
            for s in range(NF * D_F // L):
                row = rb + 2 * s + i_d8
                pu = plsc.load_gather(uf_rows, [row, i_m8])
                pv = plsc.load_gather(if_rows, [row, i_m8])
                acc += pu * pv
            tot = plsc.cumsum(acc) + plsc.load_gather(bsum, [zeros + e])
            plsc.store_scatter(out_v, [zeros + (e0 + e)], tot, mask=m15)
            return 0

        lax.fori_loop(0, CB, dot_body, 0)

    copies_prev = build_and_fire(0)
    for g in range(NB):
        copies_next = build_and_fire(g + 1) if g + 1 < NB else None
        for c in copies_prev:
            c.wait()
        compute(g)
        copies_prev = copies_next

    pltpu.sync_copy(out_v, out_hbm.at[pl.ds(base, CPW)])


@functools.partial(jax.jit, static_argnames=())
def kernel(users_features, items_features, user_id_table, user_feat_tables,
           user_bias, item_id_table, item_feat_tables, item_bias):
    ufeat8 = user_feat_tables.reshape(NF * V, D_F)
    ifeat8 = item_feat_tables.reshape(NF * V, D_F)
    brows = (V + 127) // 128
    pad = brows * 128 - user_bias.shape[0]
    ub128 = jnp.pad(user_bias.reshape(-1), (0, pad)).reshape(brows, 128)
    ib128 = jnp.pad(item_bias.reshape(-1), (0, pad)).reshape(brows, 128)
    mesh = plsc.VectorSubcoreMesh(core_axis_name="c", subcore_axis_name="s")
    per_parity = [
        pltpu.VMEM((CB,), jnp.int32),
        pltpu.VMEM((CB,), jnp.int32),
        pltpu.VMEM((CB,), jnp.int32),
        pltpu.VMEM((CB,), jnp.int32),
        pltpu.VMEM((CB,), jnp.int32),
        pltpu.VMEM((CB,), jnp.int32),
        pltpu.VMEM((FPB,), jnp.int32),
        pltpu.VMEM((FPB,), jnp.int32),
        pltpu.VMEM((FPB,), jnp.int32),
        pltpu.VMEM((FPB,), jnp.int32),
        pltpu.VMEM((CB, D_ID), jnp.float32),
        pltpu.VMEM((CB, D_ID), jnp.float32),
        pltpu.VMEM((CB, 128), jnp.float32),
        pltpu.VMEM((CB, 128), jnp.float32),
        pltpu.VMEM((FPB, D_F), jnp.float32),
        pltpu.VMEM((FPB, D_F), jnp.float32),
    ]
    f = pl.kernel(
        _nes_body,
        out_type=jax.ShapeDtypeStruct((B,), jnp.float32),
        mesh=mesh,
        scratch_types=(
            [pltpu.VMEM((CPW, NCOL), jnp.int32),
             pltpu.VMEM((CPW, NCOL), jnp.int32)]
            + per_parity + per_parity
            + [pltpu.VMEM((CB,), jnp.float32),
               pltpu.VMEM((CPW,), jnp.float32),
               pltpu.SemaphoreType.DMA,
               pltpu.SemaphoreType.DMA]
        ),
        compiler_params=pltpu.CompilerParams(
            needs_layout_passes=False, use_tc_tiling_on_sc=False),
    )
    return f(users_features, items_features, user_id_table, ub128,
             ufeat8, item_id_table, ib128, ifeat8)
